# baseline (speedup 1.0000x reference)
import jax
import jax.numpy as jnp
from jax import lax
from jax.experimental import pallas as pl
from jax.experimental.pallas import tpu as pltpu

T, K, D, F, E = 32, 8, 1024, 512, 64

EB = 2
S = E // EB
NB = 4


def _moe_kernel(ids_ref, w_ref, x_ref, wg_hbm, wu_hbm, wd_hbm, o_ref,
                wg_buf, wu_buf, wd_buf, sems):

    def _copies(s, slot):
        return (
            pltpu.make_async_copy(wg_hbm.at[pl.ds(s * EB, EB)],
                                  wg_buf.at[slot], sems.at[slot, 0]),
            pltpu.make_async_copy(wu_hbm.at[pl.ds(s * EB, EB)],
                                  wu_buf.at[slot], sems.at[slot, 1]),
            pltpu.make_async_copy(wd_hbm.at[pl.ds(s * EB, EB)],
                                  wd_buf.at[slot], sems.at[slot, 2]),
        )

    def _start(s, slot):
        for cp in _copies(s, slot):
            cp.start()

    for s in range(NB - 1):
        _start(s, s)

    x = x_ref[...]
    ids = ids_ref[...]
    w = w_ref[...]

    def body(s, acc):
        slot = lax.rem(s, NB)
        for cp in _copies(s, slot):
            cp.wait()

        nxt = s + NB - 1

        @pl.when(nxt < S)
        def _prefetch():
            _start(nxt, lax.rem(nxt, NB))

        for j in range(EB):
            e = s * EB + j
            mask = (ids == e).astype(jnp.float32)
            c = jnp.sum(w * mask, axis=1)
            g = lax.dot_general(x, wg_buf[slot, j], (((1,), (1,)), ((), ())),
                                preferred_element_type=jnp.float32)
            u = lax.dot_general(x, wu_buf[slot, j], (((1,), (1,)), ((), ())),
                                preferred_element_type=jnp.float32)
            act = (g * jax.nn.sigmoid(g)) * u
            eo = lax.dot_general(act, wd_buf[slot, j], (((1,), (1,)), ((), ())),
                                 preferred_element_type=jnp.float32)
            acc = acc + eo * c[:, None]
        return acc

    o_ref[...] = lax.fori_loop(0, S, body, jnp.zeros((T, D), jnp.float32))


def kernel(hidden_states, top_k_index, top_k_weights, w_gate, w_up, w_down):
    out = pl.pallas_call(
        _moe_kernel,
        in_specs=[
            pl.BlockSpec(memory_space=pltpu.VMEM),
            pl.BlockSpec(memory_space=pltpu.VMEM),
            pl.BlockSpec(memory_space=pltpu.VMEM),
            pl.BlockSpec(memory_space=pl.ANY),
            pl.BlockSpec(memory_space=pl.ANY),
            pl.BlockSpec(memory_space=pl.ANY),
        ],
        out_specs=pl.BlockSpec(memory_space=pltpu.VMEM),
        out_shape=jax.ShapeDtypeStruct((T, D), jnp.float32),
        scratch_shapes=[
            pltpu.VMEM((NB, EB, F, D), jnp.float32),
            pltpu.VMEM((NB, EB, F, D), jnp.float32),
            pltpu.VMEM((NB, EB, D, F), jnp.float32),
            pltpu.SemaphoreType.DMA((NB, 3)),
        ],
    )(top_k_index, top_k_weights, hidden_states, w_gate, w_up, w_down)
    return out


# speedup vs baseline: 1.0032x; 1.0032x over previous
import jax
import jax.numpy as jnp
from jax import lax
from jax.experimental import pallas as pl
from jax.experimental.pallas import tpu as pltpu

T, K, D, F, E = 32, 8, 1024, 512, 64

EB = 2
S = E // EB
NB = 3


def _moe_kernel(ids_ref, w_ref, x_ref, wg_hbm, wu_hbm, wd_hbm, o_ref,
                wg_buf, wu_buf, wd_buf, sems):

    def _copies(s, slot):
        return (
            pltpu.make_async_copy(wg_hbm.at[pl.ds(s * EB, EB)],
                                  wg_buf.at[slot], sems.at[slot, 0]),
            pltpu.make_async_copy(wu_hbm.at[pl.ds(s * EB, EB)],
                                  wu_buf.at[slot], sems.at[slot, 1]),
            pltpu.make_async_copy(wd_hbm.at[pl.ds(s * EB, EB)],
                                  wd_buf.at[slot], sems.at[slot, 2]),
        )

    def _start(s, slot):
        for cp in _copies(s, slot):
            cp.start()

    for s in range(NB - 1):
        _start(s, s)

    x = x_ref[...]
    ids = ids_ref[...]
    w = w_ref[...]

    def body(s, acc):
        slot = lax.rem(s, NB)
        for cp in _copies(s, slot):
            cp.wait()

        nxt = s + NB - 1

        @pl.when(nxt < S)
        def _prefetch():
            _start(nxt, lax.rem(nxt, NB))

        for j in range(EB):
            e = s * EB + j
            mask = (ids == e).astype(jnp.float32)
            c = jnp.sum(w * mask, axis=1)
            g = lax.dot_general(x, wg_buf[slot, j], (((1,), (1,)), ((), ())),
                                preferred_element_type=jnp.float32)
            u = lax.dot_general(x, wu_buf[slot, j], (((1,), (1,)), ((), ())),
                                preferred_element_type=jnp.float32)
            act = (g * jax.nn.sigmoid(g)) * u
            eo = lax.dot_general(act, wd_buf[slot, j], (((1,), (1,)), ((), ())),
                                 preferred_element_type=jnp.float32)
            acc = acc + eo * c[:, None]
        return acc

    o_ref[...] = lax.fori_loop(0, S, body, jnp.zeros((T, D), jnp.float32))


def kernel(hidden_states, top_k_index, top_k_weights, w_gate, w_up, w_down):
    out = pl.pallas_call(
        _moe_kernel,
        in_specs=[
            pl.BlockSpec(memory_space=pltpu.VMEM),
            pl.BlockSpec(memory_space=pltpu.VMEM),
            pl.BlockSpec(memory_space=pltpu.VMEM),
            pl.BlockSpec(memory_space=pl.ANY),
            pl.BlockSpec(memory_space=pl.ANY),
            pl.BlockSpec(memory_space=pl.ANY),
        ],
        out_specs=pl.BlockSpec(memory_space=pltpu.VMEM),
        out_shape=jax.ShapeDtypeStruct((T, D), jnp.float32),
        scratch_shapes=[
            pltpu.VMEM((NB, EB, F, D), jnp.float32),
            pltpu.VMEM((NB, EB, F, D), jnp.float32),
            pltpu.VMEM((NB, EB, D, F), jnp.float32),
            pltpu.SemaphoreType.DMA((NB, 3)),
        ],
    )(top_k_index, top_k_weights, hidden_states, w_gate, w_up, w_down)
    return out
